# baseline (speedup 1.0000x reference)
import functools

import jax
import jax.numpy as jnp
from jax import lax
from jax.experimental import pallas as pl
from jax.experimental.pallas import tpu as pltpu
from jax.experimental.pallas import tpu_sc as plsc

B = 16384
D = 64
NC = 2
NS = 16
L = 16
NW = NC * NS
BPW = B // NW


def _body(idx0_hbm, idx1_hbm, fe0_hbm, fe1_hbm, out_hbm,
          idx0_v, idx1_v, rows_a, rows_b, out_v, sem_a, sem_b):
    wid = lax.axis_index("s") * NC + lax.axis_index("c")
    base = wid * BPW

    pltpu.sync_copy(idx0_hbm.at[pl.ds(base, BPW)], idx0_v)
    pltpu.sync_copy(idx1_hbm.at[pl.ds(base, BPW)], idx1_v)
    cp_a = pltpu.async_copy(fe0_hbm.at[idx0_v], rows_a, sem_a)
    cp_b = pltpu.async_copy(fe1_hbm.at[idx1_v], rows_b, sem_b)
    cp_a.wait()
    cp_b.wait()

    lane = lax.iota(jnp.int32, 16)

    def group(g, carry):
        rows = g * L + lane
        acc = jnp.zeros((L,), jnp.float32)
        for j in range(D):
            col = jnp.full((L,), j, jnp.int32)
            a = plsc.load_gather(rows_a, [rows, col])
            b = plsc.load_gather(rows_b, [rows, col])
            acc = acc + a * b
        res = 1.0 / (1.0 + jnp.exp(-acc))
        out_v[pl.ds(g * L, L)] = res
        return carry

    lax.fori_loop(0, BPW // L, group, 0)
    pltpu.sync_copy(out_v, out_hbm.at[pl.ds(base, BPW)])


@jax.jit
def kernel(xs, fe_0, fe_1):
    idx = xs.astype(jnp.int32)
    idx0 = idx[:, 0]
    idx1 = idx[:, 1]

    mesh = plsc.VectorSubcoreMesh(core_axis_name="c", subcore_axis_name="s")
    run = pl.kernel(
        _body,
        out_type=jax.ShapeDtypeStruct((B,), jnp.float32),
        mesh=mesh,
        compiler_params=pltpu.CompilerParams(
            needs_layout_passes=False,
            use_tc_tiling_on_sc=False,
        ),
        scratch_types=[
            pltpu.VMEM((BPW,), jnp.int32),
            pltpu.VMEM((BPW,), jnp.int32),
            pltpu.VMEM((BPW, D), jnp.float32),
            pltpu.VMEM((BPW, D), jnp.float32),
            pltpu.VMEM((BPW,), jnp.float32),
            pltpu.SemaphoreType.DMA,
            pltpu.SemaphoreType.DMA,
        ],
    )
    return run(idx0, idx1, fe_0, fe_1)


# speedup vs baseline: 1.5116x; 1.5116x over previous
import jax
import jax.numpy as jnp
from jax import lax
from jax.experimental import pallas as pl
from jax.experimental.pallas import tpu as pltpu
from jax.experimental.pallas import tpu_sc as plsc

B = 16384
D = 64
NC = 2
NS = 16
L = 16
NW = NC * NS
BPW = B // NW
CHUNK = 128


def _body(idx0_hbm, idx1_hbm, fe0_hbm, fe1_hbm, out_hbm,
          idx0_v, idx1_v, rows_a, rows_b, out_v, sem_a, sem_b):
    wid = lax.axis_index("s") * NC + lax.axis_index("c")
    base = wid * BPW

    pltpu.sync_copy(idx0_hbm.at[pl.ds(base, BPW)], idx0_v)
    pltpu.sync_copy(idx1_hbm.at[pl.ds(base, BPW)], idx1_v)

    lane = lax.iota(jnp.int32, L)

    def chunk(c, carry):
        c0 = c * CHUNK

        def fetch(g, carry2):
            k0 = c0 + g * L
            tv0 = idx0_v[pl.ds(k0, L)]
            tv1 = idx1_v[pl.ds(k0, L)]
            cps = []
            for e in range(L):
                cps.append(pltpu.async_copy(
                    fe0_hbm.at[tv0[e]], rows_a.at[g * L + e], sem_a))
                cps.append(pltpu.async_copy(
                    fe1_hbm.at[tv1[e]], rows_b.at[g * L + e], sem_b))
            for cp in cps:
                cp.wait()
            return carry2

        lax.fori_loop(0, CHUNK // L, fetch, 0)

        def group(g, carry2):
            rows = g * L + lane
            acc = jnp.zeros((L,), jnp.float32)
            for j in range(D):
                col = jnp.full((L,), j, jnp.int32)
                a = plsc.load_gather(rows_a, [rows, col])
                b = plsc.load_gather(rows_b, [rows, col])
                acc = acc + a * b
            res = 1.0 / (1.0 + jnp.exp(-acc))
            out_v[pl.ds(c0 + g * L, L)] = res
            return carry2

        lax.fori_loop(0, CHUNK // L, group, 0)
        return carry

    lax.fori_loop(0, BPW // CHUNK, chunk, 0)
    pltpu.sync_copy(out_v, out_hbm.at[pl.ds(base, BPW)])


@jax.jit
def kernel(xs, fe_0, fe_1):
    idx = xs.astype(jnp.int32)
    idx0 = idx[:, 0]
    idx1 = idx[:, 1]

    mesh = plsc.VectorSubcoreMesh(core_axis_name="c", subcore_axis_name="s")
    run = pl.kernel(
        _body,
        out_type=jax.ShapeDtypeStruct((B,), jnp.float32),
        mesh=mesh,
        compiler_params=pltpu.CompilerParams(
            needs_layout_passes=False,
            use_tc_tiling_on_sc=True,
        ),
        scratch_types=[
            pltpu.VMEM((BPW,), jnp.int32),
            pltpu.VMEM((BPW,), jnp.int32),
            pltpu.VMEM((CHUNK, D), jnp.float32),
            pltpu.VMEM((CHUNK, D), jnp.float32),
            pltpu.VMEM((BPW,), jnp.float32),
            pltpu.SemaphoreType.DMA,
            pltpu.SemaphoreType.DMA,
        ],
    )
    return run(idx0, idx1, fe_0, fe_1)


# speedup vs baseline: 2.6317x; 1.7410x over previous
import jax
import jax.numpy as jnp
from jax import lax
from jax.experimental import pallas as pl
from jax.experimental.pallas import tpu as pltpu
from jax.experimental.pallas import tpu_sc as plsc

B = 16384
D = 64
NC = 2
NS = 16
L = 16
NW = NC * NS
BPW = B // NW
WIN = 128
DEPTH = 4
NG = BPW // L


def _extract(win_ref, col, dst_ref, e, lane):
    cv = jnp.full((L,), col, jnp.int32)
    ev = jnp.full((L,), e, jnp.int32)
    for q in range(D // L):
        jv = q * L + lane
        v = plsc.load_gather(win_ref, [jv, cv])
        plsc.store_scatter(dst_ref, [jv, ev], v)


def _body(idx0_hbm, idx1_hbm, fe0t_hbm, fe1t_hbm, out_hbm,
          idx0_v, idx1_v, wa0, wa1, wa2, wa3, wb0, wb1, wb2, wb3,
          ga0, gb0, ga1, gb1, out_v,
          sa0, sa1, sa2, sa3, sb0, sb1, sb2, sb3):
    wid = lax.axis_index("s") * NC + lax.axis_index("c")
    base = wid * BPW

    pltpu.sync_copy(idx0_hbm.at[pl.ds(base, BPW)], idx0_v)
    pltpu.sync_copy(idx1_hbm.at[pl.ds(base, BPW)], idx1_v)

    lane = lax.iota(jnp.int32, L)
    wa = [wa0, wa1, wa2, wa3]
    wb = [wb0, wb1, wb2, wb3]
    sa = [sa0, sa1, sa2, sa3]
    sb = [sb0, sb1, sb2, sb3]
    gab = [(ga0, gb0), (ga1, gb1)]

    def dot_group(ga, gb, k0):
        acc = jnp.zeros((L,), jnp.float32)
        for j in range(D):
            jv = jnp.full((L,), j, jnp.int32)
            a = plsc.load_gather(ga, [jv, lane])
            b = plsc.load_gather(gb, [jv, lane])
            acc = acc + a * b
        res = 1.0 / (1.0 + jnp.exp(-acc))
        out_v[pl.ds(k0, L)] = res

    def pair(t, carry):
        for sub in range(2):
            g = t * 2 + sub
            ga, gb = gab[sub]
            pga, pgb = gab[1 - sub]
            k0 = g * L
            iv0 = idx0_v[pl.ds(k0, L)]
            iv1 = idx1_v[pl.ds(k0, L)]
            cv0 = iv0 & (WIN - 1)
            cv1 = iv1 & (WIN - 1)

            cps = [None] * L
            for e in range(DEPTH):
                s = e % DEPTH
                w0 = pl.multiple_of(iv0[e] & ~(WIN - 1), WIN)
                w1 = pl.multiple_of(iv1[e] & ~(WIN - 1), WIN)
                cps[e] = (
                    pltpu.async_copy(
                        fe0t_hbm.at[:, pl.ds(w0, WIN)], wa[s], sa[s]),
                    pltpu.async_copy(
                        fe1t_hbm.at[:, pl.ds(w1, WIN)], wb[s], sb[s]),
                )

            if sub == 1:
                dot_group(pga, pgb, k0 - L)
            else:
                @pl.when(g > 0)
                def _():
                    dot_group(pga, pgb, k0 - L)

            for e in range(DEPTH, L):
                s = e % DEPTH
                pa, pb = cps[e - DEPTH]
                pa.wait()
                _extract(wa[s], cv0[e - DEPTH], ga, e - DEPTH, lane)
                pb.wait()
                _extract(wb[s], cv1[e - DEPTH], gb, e - DEPTH, lane)
                w0 = pl.multiple_of(iv0[e] & ~(WIN - 1), WIN)
                w1 = pl.multiple_of(iv1[e] & ~(WIN - 1), WIN)
                cps[e] = (
                    pltpu.async_copy(
                        fe0t_hbm.at[:, pl.ds(w0, WIN)], wa[s], sa[s]),
                    pltpu.async_copy(
                        fe1t_hbm.at[:, pl.ds(w1, WIN)], wb[s], sb[s]),
                )
            for e in range(L - DEPTH, L):
                s = e % DEPTH
                pa, pb = cps[e]
                pa.wait()
                _extract(wa[s], cv0[e], ga, e, lane)
                pb.wait()
                _extract(wb[s], cv1[e], gb, e, lane)
        return carry

    lax.fori_loop(0, NG // 2, pair, 0)
    dot_group(*gab[1], (NG - 1) * L)
    pltpu.sync_copy(out_v, out_hbm.at[pl.ds(base, BPW)])


def _impl(xs, fe_0, fe_1):
    idx = xs.astype(jnp.int32)
    idx0 = idx[:, 0]
    idx1 = idx[:, 1]
    fe0t = fe_0.T
    fe1t = fe_1.T

    mesh = plsc.VectorSubcoreMesh(core_axis_name="c", subcore_axis_name="s")
    run = pl.kernel(
        _body,
        out_type=jax.ShapeDtypeStruct((B,), jnp.float32),
        mesh=mesh,
        compiler_params=pltpu.CompilerParams(
            needs_layout_passes=False,
            use_tc_tiling_on_sc=True,
        ),
        scratch_types=(
            [pltpu.VMEM((BPW,), jnp.int32)] * 2
            + [pltpu.VMEM((D, WIN), jnp.float32)] * (2 * DEPTH)
            + [pltpu.VMEM((D, L), jnp.float32)] * 4
            + [pltpu.VMEM((BPW,), jnp.float32)]
            + [pltpu.SemaphoreType.DMA] * (2 * DEPTH)
        ),
    )
    return run(idx0, idx1, fe0t, fe1t)


kernel = jax.jit(_impl)
